# Initial kernel scaffold; baseline (speedup 1.0000x reference)
import jax
import jax.numpy as jnp
from jax.experimental import pallas as pl


def kernel(x_tokens, y_tokens, x_table, y_table, pos_table):
    raise NotImplementedError("write your pallas kernel here")


# speedup vs baseline: 5.8839x; 5.8839x over previous
import functools

import jax
import jax.numpy as jnp
from jax import lax
from jax.experimental import pallas as pl
from jax.experimental.pallas import tpu as pltpu
from jax.experimental.pallas import tpu_sc as plsc

NBINS_X = 1000
MAX_Y_TOKENS = 201
EMBED_DIM = 64
MAX_LEN = 200
B = 4096
T = 200
N = B * T

CHUNK = 400
POS_REP = CHUNK // MAX_LEN


def _make_kernel():
    info = plsc.get_sparse_core_info()
    nc, ns = info.num_cores, info.num_subcores
    nw = nc * ns
    per_w = N // nw
    n_chunks = per_w // CHUNK
    assert per_w % CHUNK == 0

    mesh = plsc.VectorSubcoreMesh(core_axis_name="c", subcore_axis_name="s")

    @functools.partial(
        pl.kernel,
        mesh=mesh,
        out_type=jax.ShapeDtypeStruct((N, EMBED_DIM), jnp.float32),
        compiler_params=pltpu.CompilerParams(use_tc_tiling_on_sc=False),
        scratch_types=[
            pltpu.VMEM((CHUNK,), jnp.int32),
            pltpu.VMEM((CHUNK,), jnp.int32),
            pltpu.VMEM((CHUNK, EMBED_DIM), jnp.float32),
            pltpu.VMEM((CHUNK, EMBED_DIM), jnp.float32),
            pltpu.VMEM((CHUNK, EMBED_DIM), jnp.float32),
            pltpu.SemaphoreType.DMA,
            pltpu.SemaphoreType.DMA,
        ],
    )
    def k(x_tok, y_tok, x_table, y_table, pos_table, out,
          idx_x, idx_y, acc, yrows, pos_v, sem_x, sem_y):
        wid = lax.axis_index("s") * nc + lax.axis_index("c")
        w_base = wid * per_w

        for r in range(POS_REP):
            pltpu.sync_copy(pos_table, pos_v.at[pl.ds(r * MAX_LEN, MAX_LEN)])

        def step(g, carry):
            base = w_base + g * CHUNK
            pltpu.sync_copy(x_tok.at[pl.ds(base, CHUNK)], idx_x)
            pltpu.sync_copy(y_tok.at[pl.ds(base, CHUNK)], idx_y)
            cx = pltpu.async_copy(x_table.at[idx_x], acc, sem_x)
            cy = pltpu.async_copy(y_table.at[idx_y], yrows, sem_y)
            cx.wait()
            cy.wait()

            def add_row(i, c):
                for j in range(EMBED_DIM // 16):
                    ds = pl.ds(j * 16, 16)
                    acc[i, ds] = acc[i, ds] + yrows[i, ds] + pos_v[i, ds]
                return c

            lax.fori_loop(0, CHUNK, add_row, 0, unroll=2)
            pltpu.sync_copy(acc, out.at[pl.ds(base, CHUNK)])
            return carry

        lax.fori_loop(0, n_chunks, step, 0)

    return k


_sc_kernel = _make_kernel()


def kernel(x_tokens, y_tokens, x_table, y_table, pos_table):
    xt = x_tokens.reshape(N).astype(jnp.int32)
    yt = y_tokens.reshape(N).astype(jnp.int32)
    out = _sc_kernel(xt, yt, x_table, y_table, pos_table)
    return out.reshape(B, T, EMBED_DIM)


# speedup vs baseline: 7.2133x; 1.2259x over previous
import functools

import jax
import jax.numpy as jnp
from jax import lax
from jax.experimental import pallas as pl
from jax.experimental.pallas import tpu as pltpu
from jax.experimental.pallas import tpu_sc as plsc

NBINS_X = 1000
MAX_Y_TOKENS = 201
EMBED_DIM = 64
MAX_LEN = 200
B = 4096
T = 200
N = B * T

CHUNK = 200


def _make_kernel():
    info = plsc.get_sparse_core_info()
    nc, ns = info.num_cores, info.num_subcores
    nw = nc * ns
    per_w = N // nw
    n_chunks = per_w // CHUNK
    assert per_w % CHUNK == 0 and n_chunks % 2 == 0 and n_chunks >= 6

    mesh = plsc.VectorSubcoreMesh(core_axis_name="c", subcore_axis_name="s")

    f32 = jnp.float32
    i32 = jnp.int32

    @functools.partial(
        pl.kernel,
        mesh=mesh,
        out_type=jax.ShapeDtypeStruct((N, EMBED_DIM), f32),
        compiler_params=pltpu.CompilerParams(use_tc_tiling_on_sc=False),
        scratch_types=[
            pltpu.VMEM((CHUNK,), i32), pltpu.VMEM((CHUNK,), i32),
            pltpu.VMEM((CHUNK,), i32), pltpu.VMEM((CHUNK,), i32),
            pltpu.VMEM((CHUNK, EMBED_DIM), f32), pltpu.VMEM((CHUNK, EMBED_DIM), f32),
            pltpu.VMEM((CHUNK, EMBED_DIM), f32), pltpu.VMEM((CHUNK, EMBED_DIM), f32),
            pltpu.VMEM((CHUNK, EMBED_DIM), f32), pltpu.VMEM((CHUNK, EMBED_DIM), f32),
            pltpu.VMEM((CHUNK, EMBED_DIM), f32),
            pltpu.SemaphoreType.DMA, pltpu.SemaphoreType.DMA,
            pltpu.SemaphoreType.DMA, pltpu.SemaphoreType.DMA,
            pltpu.SemaphoreType.DMA, pltpu.SemaphoreType.DMA,
            pltpu.SemaphoreType.DMA, pltpu.SemaphoreType.DMA,
        ],
    )
    def k(x_tok, y_tok, x_table, y_table, pos_table, out,
          ix_a, ix_b, iy_a, iy_b, xr_a, xr_b, yr_a, yr_b, acc_a, acc_b,
          pos_v, sgx_a, sgx_b, sgy_a, sgy_b, so_a, so_b, si_a, si_b):
        wid = lax.axis_index("s") * nc + lax.axis_index("c")
        w_base = wid * per_w

        ix = (ix_a, ix_b)
        iy = (iy_a, iy_b)
        xr = (xr_a, xr_b)
        yr = (yr_a, yr_b)
        acc = (acc_a, acc_b)
        sgx = (sgx_a, sgx_b)
        sgy = (sgy_a, sgy_b)
        so = (so_a, so_b)
        si = (si_a, si_b)

        pltpu.sync_copy(pos_table, pos_v)

        def start_gathers(p):
            pltpu.async_copy(x_table.at[ix[p]], xr[p], sgx[p])
            pltpu.async_copy(y_table.at[iy[p]], yr[p], sgy[p])

        def wait_gathers(p):
            pltpu.make_async_copy(x_table.at[ix[p]], xr[p], sgx[p]).wait()
            pltpu.make_async_copy(y_table.at[iy[p]], yr[p], sgy[p]).wait()

        def start_idx_load(base, p):
            pltpu.async_copy(x_tok.at[pl.ds(base, CHUNK)], ix[p], si[p])
            pltpu.async_copy(y_tok.at[pl.ds(base, CHUNK)], iy[p], si[p])

        def wait_idx_load(p):
            pltpu.make_async_copy(x_tok.at[pl.ds(0, CHUNK)], ix[p], si[p]).wait()
            pltpu.make_async_copy(y_tok.at[pl.ds(0, CHUNK)], iy[p], si[p]).wait()

        def compute(p):
            xp, yp, ap = xr[p], yr[p], acc[p]

            def add_row(i, c):
                for j in range(EMBED_DIM // 16):
                    ds = pl.ds(j * 16, 16)
                    ap[i, ds] = xp[i, ds] + yp[i, ds] + pos_v[i, ds]
                return c

            lax.fori_loop(0, CHUNK, add_row, 0, unroll=2)

        def start_out(base, p):
            pltpu.async_copy(acc[p], out.at[pl.ds(base, CHUNK)], so[p])

        def wait_out(p):
            pltpu.make_async_copy(acc[p], out.at[pl.ds(0, CHUNK)], so[p]).wait()

        pltpu.sync_copy(x_tok.at[pl.ds(w_base, CHUNK)], ix_a)
        pltpu.sync_copy(y_tok.at[pl.ds(w_base, CHUNK)], iy_a)
        start_gathers(0)
        pltpu.sync_copy(x_tok.at[pl.ds(w_base + CHUNK, CHUNK)], ix_b)
        pltpu.sync_copy(y_tok.at[pl.ds(w_base + CHUNK, CHUNK)], iy_b)

        wait_gathers(0)
        start_idx_load(w_base + 2 * CHUNK, 0)
        start_gathers(1)
        compute(0)
        start_out(w_base, 0)

        wait_gathers(1)
        start_idx_load(w_base + 3 * CHUNK, 1)
        wait_idx_load(0)
        start_gathers(0)
        compute(1)
        start_out(w_base + CHUNK, 1)

        def pair(j, carry):
            for p in range(2):
                g_base = w_base + (2 * j + p) * CHUNK
                wait_gathers(p)
                start_idx_load(g_base + 2 * CHUNK, p)
                wait_idx_load(1 - p)
                start_gathers(1 - p)
                wait_out(p)
                compute(p)
                start_out(g_base, p)
            return carry

        lax.fori_loop(1, n_chunks // 2 - 1, pair, 0)

        base = w_base + (n_chunks - 2) * CHUNK
        wait_gathers(0)
        wait_idx_load(1)
        start_gathers(1)
        wait_out(0)
        compute(0)
        start_out(base, 0)

        wait_gathers(1)
        wait_out(1)
        compute(1)
        start_out(base + CHUNK, 1)

        wait_out(0)
        wait_out(1)

    return k


_sc_kernel = _make_kernel()


def kernel(x_tokens, y_tokens, x_table, y_table, pos_table):
    xt = x_tokens.reshape(N).astype(jnp.int32)
    yt = y_tokens.reshape(N).astype(jnp.int32)
    out = _sc_kernel(xt, yt, x_table, y_table, pos_table)
    return out.reshape(B, T, EMBED_DIM)


# speedup vs baseline: 7.2272x; 1.0019x over previous
import functools

import jax
import jax.numpy as jnp
from jax import lax
from jax.experimental import pallas as pl
from jax.experimental.pallas import tpu as pltpu
from jax.experimental.pallas import tpu_sc as plsc

NBINS_X = 1000
MAX_Y_TOKENS = 201
EMBED_DIM = 64
MAX_LEN = 200
B = 4096
T = 200
N = B * T

CHUNK = 200


def _make_kernel():
    info = plsc.get_sparse_core_info()
    nc, ns = info.num_cores, info.num_subcores
    nw = nc * ns
    n_chunks = B // nw
    assert B % nw == 0 and n_chunks % 2 == 0 and n_chunks >= 6

    mesh = plsc.VectorSubcoreMesh(core_axis_name="c", subcore_axis_name="s")

    f32 = jnp.float32
    i32 = jnp.int32

    @functools.partial(
        pl.kernel,
        mesh=mesh,
        out_type=jax.ShapeDtypeStruct((B, T, EMBED_DIM), f32),
        compiler_params=pltpu.CompilerParams(use_tc_tiling_on_sc=False),
        scratch_types=[
            pltpu.VMEM((CHUNK,), i32), pltpu.VMEM((CHUNK,), i32),
            pltpu.VMEM((CHUNK,), i32), pltpu.VMEM((CHUNK,), i32),
            pltpu.VMEM((CHUNK, EMBED_DIM), f32), pltpu.VMEM((CHUNK, EMBED_DIM), f32),
            pltpu.VMEM((CHUNK, EMBED_DIM), f32), pltpu.VMEM((CHUNK, EMBED_DIM), f32),
            pltpu.VMEM((CHUNK, EMBED_DIM), f32), pltpu.VMEM((CHUNK, EMBED_DIM), f32),
            pltpu.VMEM((CHUNK, EMBED_DIM), f32),
            pltpu.SemaphoreType.DMA, pltpu.SemaphoreType.DMA,
            pltpu.SemaphoreType.DMA, pltpu.SemaphoreType.DMA,
            pltpu.SemaphoreType.DMA, pltpu.SemaphoreType.DMA,
            pltpu.SemaphoreType.DMA, pltpu.SemaphoreType.DMA,
        ],
    )
    def k(x_tok, y_tok, x_table, y_table, pos_table, out,
          ix_a, ix_b, iy_a, iy_b, xr_a, xr_b, yr_a, yr_b, acc_a, acc_b,
          pos_v, sgx_a, sgx_b, sgy_a, sgy_b, so_a, so_b, si_a, si_b):
        wid = lax.axis_index("s") * nc + lax.axis_index("c")
        w_row = wid * n_chunks

        ix = (ix_a, ix_b)
        iy = (iy_a, iy_b)
        xr = (xr_a, xr_b)
        yr = (yr_a, yr_b)
        acc = (acc_a, acc_b)
        sgx = (sgx_a, sgx_b)
        sgy = (sgy_a, sgy_b)
        so = (so_a, so_b)
        si = (si_a, si_b)

        pltpu.sync_copy(pos_table, pos_v)

        def start_gathers(p):
            pltpu.async_copy(x_table.at[ix[p]], xr[p], sgx[p])
            pltpu.async_copy(y_table.at[iy[p]], yr[p], sgy[p])

        def wait_gathers(p):
            pltpu.make_async_copy(x_table.at[ix[p]], xr[p], sgx[p]).wait()
            pltpu.make_async_copy(y_table.at[iy[p]], yr[p], sgy[p]).wait()

        def start_idx_load(row, p):
            pltpu.async_copy(x_tok.at[row], ix[p], si[p])
            pltpu.async_copy(y_tok.at[row], iy[p], si[p])

        def wait_idx_load(p):
            pltpu.make_async_copy(x_tok.at[0], ix[p], si[p]).wait()
            pltpu.make_async_copy(y_tok.at[0], iy[p], si[p]).wait()

        def compute(p):
            xp, yp, ap = xr[p], yr[p], acc[p]

            def add_row(i, c):
                for j in range(EMBED_DIM // 16):
                    ds = pl.ds(j * 16, 16)
                    ap[i, ds] = xp[i, ds] + yp[i, ds] + pos_v[i, ds]
                return c

            lax.fori_loop(0, CHUNK, add_row, 0, unroll=2)

        def start_out(row, p):
            pltpu.async_copy(acc[p], out.at[row], so[p])

        def wait_out(p):
            pltpu.make_async_copy(acc[p], out.at[0], so[p]).wait()

        pltpu.sync_copy(x_tok.at[w_row], ix_a)
        pltpu.sync_copy(y_tok.at[w_row], iy_a)
        start_gathers(0)
        pltpu.sync_copy(x_tok.at[w_row + 1], ix_b)
        pltpu.sync_copy(y_tok.at[w_row + 1], iy_b)

        wait_gathers(0)
        start_idx_load(w_row + 2, 0)
        start_gathers(1)
        compute(0)
        start_out(w_row, 0)

        wait_gathers(1)
        start_idx_load(w_row + 3, 1)
        wait_idx_load(0)
        start_gathers(0)
        compute(1)
        start_out(w_row + 1, 1)

        def pair(j, carry):
            for p in range(2):
                row = w_row + 2 * j + p
                wait_gathers(p)
                start_idx_load(row + 2, p)
                wait_idx_load(1 - p)
                start_gathers(1 - p)
                wait_out(p)
                compute(p)
                start_out(row, p)
            return carry

        lax.fori_loop(1, n_chunks // 2 - 1, pair, 0)

        row = w_row + n_chunks - 2
        wait_gathers(0)
        wait_idx_load(1)
        start_gathers(1)
        wait_out(0)
        compute(0)
        start_out(row, 0)

        wait_gathers(1)
        wait_out(1)
        compute(1)
        start_out(row + 1, 1)

        wait_out(0)
        wait_out(1)

    return k


_sc_kernel = _make_kernel()


def kernel(x_tokens, y_tokens, x_table, y_table, pos_table):
    return _sc_kernel(x_tokens.astype(jnp.int32), y_tokens.astype(jnp.int32),
                      x_table, y_table, pos_table)
